# Initial kernel scaffold; baseline (speedup 1.0000x reference)
import jax
import jax.numpy as jnp
from jax.experimental import pallas as pl


def kernel(X, B, C_unary, C_pair, C):
    raise NotImplementedError("write your pallas kernel here")


# speedup vs baseline: 10.5794x; 10.5794x over previous
import functools

import jax
import jax.numpy as jnp
from jax import lax
from jax.experimental import pallas as pl
from jax.experimental.pallas import tpu as pltpu
from jax.experimental.pallas import tpu_sc as plsc

_NC = 2
_NS = 16
_NW = _NC * _NS
_L = 16


def _base_kernel_body(x_ref, c_ref, u_ref, o_ref):
    x = x_ref[...]
    c = c_ref[...]
    acc = lax.dot_general(x, c, (((1,), (1,)), ((), ())),
                          preferred_element_type=jnp.float32)
    o_ref[...] = -2.0 * acc + u_ref[...]


def _make_base(N, M, K, D, BN):
    nb = N // BN
    return pl.pallas_call(
        _base_kernel_body,
        grid=(M, nb),
        in_specs=[
            pl.BlockSpec((BN, D), lambda i, b: (b, 0)),
            pl.BlockSpec((K, D), lambda i, b: (i, 0)),
            pl.BlockSpec((BN, K), lambda i, b: (b, i)),
        ],
        out_specs=pl.BlockSpec((BN, K), lambda i, b: (i * nb + b, 0)),
        out_shape=jax.ShapeDtypeStruct((M * N, K), jnp.float32),
    )


def _transpose_body(i_ref, o_ref):
    o_ref[...] = i_ref[0].T


def _make_transpose(MM, K):
    return pl.pallas_call(
        _transpose_body,
        grid=(MM,),
        in_specs=[pl.BlockSpec((1, K, K), lambda g: (g, 0, 0))],
        out_specs=pl.BlockSpec((K, K), lambda g: (g, 0)),
        out_shape=jax.ShapeDtypeStruct((MM * K, K), jnp.float32),
    )


def _make_icm(N, M, K, n_iter, rows_per_w, ch):
    nch = rows_per_w // ch
    nidx = ch * M
    mesh = plsc.VectorSubcoreMesh(core_axis_name="c", subcore_axis_name="s")

    @functools.partial(
        pl.kernel,
        mesh=mesh,
        out_type=jax.ShapeDtypeStruct((M * N,), jnp.int32),
        scratch_types=[
            pltpu.VMEM((M * rows_per_w,), jnp.int32),
            pltpu.VMEM((nidx,), jnp.int32),
            pltpu.VMEM((nidx, K), jnp.float32),
            pltpu.VMEM((ch, K), jnp.float32),
            pltpu.SemaphoreType.DMA,
        ],
    )
    def icm(base_hbm, b_hbm, tab_hbm, out_hbm, b_loc, idx_v, g_v, base_v, sem):
        wid = lax.axis_index("s") * _NC + lax.axis_index("c")
        n0 = wid * rows_per_w
        iota = lax.iota(jnp.int32, _L)
        inf16 = jnp.broadcast_to(jnp.float32(jnp.inf), (_L,))
        zero16 = jnp.broadcast_to(jnp.int32(0), (_L,))

        for j in range(M):
            pltpu.sync_copy(b_hbm.at[pl.ds(j * N + n0, rows_per_w)],
                            b_loc.at[pl.ds(j * rows_per_w, rows_per_w)])

        def sweep_body(t, carry):
            i = t & (M - 1)
            tab_off = i * (M * K)

            def chunk_body(cc, carry2):
                row0 = cc * ch
                for j in range(M):
                    bv = b_loc[pl.ds(j * rows_per_w + row0, _L)]
                    idx_v[pl.ds(j * _L, _L)] = bv + (tab_off + j * K)
                pltpu.async_copy(tab_hbm.at[idx_v], g_v, sem).wait()
                pltpu.sync_copy(
                    base_hbm.at[pl.ds(i * N + n0 + row0, ch)], base_v)

                def n_body(nn, res):
                    def s_body(sg, mc):
                        mv, mi = mc
                        a = base_v[nn, pl.ds(sg * _L, _L)]
                        for j in range(M):
                            a = a + g_v[j * ch + nn, pl.ds(sg * _L, _L)]
                        kv = sg * _L + iota
                        p = a < mv
                        return (jnp.where(p, a, mv), jnp.where(p, kv, mi))

                    mv, mi = lax.fori_loop(0, K // _L, s_body,
                                           (inf16, zero16))
                    for sh in (8, 4, 2, 1):
                        perm = iota ^ sh
                        mv2 = mv.at[perm].get(mode="promise_in_bounds")
                        mi2 = mi.at[perm].get(mode="promise_in_bounds")
                        take = (mv2 < mv) | ((mv2 == mv) & (mi2 < mi))
                        mv = jnp.where(take, mv2, mv)
                        mi = jnp.where(take, mi2, mi)
                    return jnp.where(iota == nn, mi, res)

                res = lax.fori_loop(0, ch, n_body, zero16)
                b_loc[pl.ds(i * rows_per_w + row0, _L)] = res
                return carry2

            return lax.fori_loop(0, nch, chunk_body, carry)

        lax.fori_loop(0, n_iter * M, sweep_body, 0)
        for j in range(M):
            pltpu.sync_copy(b_loc.at[pl.ds(j * rows_per_w, rows_per_w)],
                            out_hbm.at[pl.ds(j * N + n0, rows_per_w)])

    return icm


def kernel(X, B, C_unary, C_pair, C):
    N, D = X.shape
    M = B.shape[1]
    K = C_pair.shape[2]

    base = _make_base(N, M, K, D, BN=512)(X, C, C_unary)
    tab = _make_transpose(M * M, K)(C_pair.reshape(M * M, K, K))
    rows_per_w = N // _NW
    out = _make_icm(N, M, K, 2, rows_per_w, ch=_L)(
        base, B.T.reshape(-1), tab)
    return out.reshape(M, N).T


# speedup vs baseline: 19.7394x; 1.8658x over previous
import functools

import jax
import jax.numpy as jnp
from jax import lax
from jax.experimental import pallas as pl
from jax.experimental.pallas import tpu as pltpu
from jax.experimental.pallas import tpu_sc as plsc

_NC = 2
_NS = 16
_NW = _NC * _NS
_L = 16
_NPH = 4


def _make_base(N, M, K, D, BN):
    nb = N // BN
    KP = K // _NPH

    def body(x_ref, c_ref, u_ref, *o_refs):
        acc = lax.dot_general(x_ref[...], c_ref[...], (((1,), (1,)), ((), ())),
                              preferred_element_type=jnp.float32)
        val = -2.0 * acc + u_ref[...]
        for p in range(_NPH):
            o_refs[p][...] = val[:, p * KP:(p + 1) * KP]

    return pl.pallas_call(
        body,
        grid=(M, nb),
        in_specs=[
            pl.BlockSpec((BN, D), lambda i, b: (b, 0)),
            pl.BlockSpec((K, D), lambda i, b: (i, 0)),
            pl.BlockSpec((BN, K), lambda i, b: (b, i)),
        ],
        out_specs=[pl.BlockSpec((BN, KP), lambda i, b, nb=nb: (i * nb + b, 0))
                   for _ in range(_NPH)],
        out_shape=[jax.ShapeDtypeStruct((M * N, KP), jnp.float32)
                   for _ in range(_NPH)],
    )


def _make_transpose(MM, K):
    KP = K // _NPH

    def body(i_ref, *o_refs):
        x = i_ref[0]
        for p in range(_NPH):
            o_refs[p][...] = x[p * KP:(p + 1) * KP, :].T

    return pl.pallas_call(
        body,
        grid=(MM,),
        in_specs=[pl.BlockSpec((1, K, K), lambda g: (g, 0, 0))],
        out_specs=[pl.BlockSpec((K, KP), lambda g: (g, 0))
                   for _ in range(_NPH)],
        out_shape=[jax.ShapeDtypeStruct((MM * K, KP), jnp.float32)
                   for _ in range(_NPH)],
    )


def _make_icm(N, M, K, n_iter, rows_per_w):
    KP = K // _NPH
    GR = _L
    ngrp = rows_per_w // GR
    nidx = GR * M
    nsweep = n_iter * M
    mesh = plsc.VectorSubcoreMesh(core_axis_name="c", subcore_axis_name="s")

    @functools.partial(
        pl.kernel,
        mesh=mesh,
        out_type=jax.ShapeDtypeStruct((M * N,), jnp.int32),
        scratch_types=[
            pltpu.VMEM((M * rows_per_w,), jnp.int32),
            [pltpu.VMEM((nidx,), jnp.int32) for _ in range(2)],
            [pltpu.VMEM((nidx, KP), jnp.float32) for _ in range(_NPH)],
            [pltpu.VMEM((GR, KP), jnp.float32) for _ in range(_NPH)],
            pltpu.VMEM((GR, _L), jnp.float32),
            pltpu.VMEM((GR, _L), jnp.int32),
            [pltpu.SemaphoreType.DMA for _ in range(_NPH)],
            [pltpu.SemaphoreType.DMA for _ in range(_NPH)],
        ],
    )
    def icm(b0, b1, b2, b3, t0, t1, t2, t3, b_hbm, out_hbm,
            b_loc, idx2, gb, bb, mv_s, mi_s, gsem, bsem):
        bases = (b0, b1, b2, b3)
        tabs = (t0, t1, t2, t3)
        wid = lax.axis_index("s") * _NC + lax.axis_index("c")
        n0 = wid * rows_per_w
        iota = lax.iota(jnp.int32, _L)
        inf16 = jnp.broadcast_to(jnp.float32(jnp.inf), (_L,))
        zero16 = jnp.broadcast_to(jnp.int32(0), (_L,))

        for j in range(M):
            pltpu.sync_copy(b_hbm.at[pl.ds(j * N + n0, rows_per_w)],
                            b_loc.at[pl.ds(j * rows_per_w, rows_per_w)])

        def build_idx(buf, row0, toff):
            for j in range(M):
                bv = b_loc[pl.ds(j * rows_per_w + row0, _L)]
                buf[pl.ds(j * _L, _L)] = bv + (toff + j * K)

        def issue(p, buf, row0, i_nxt):
            pltpu.async_copy(tabs[p].at[buf], gb[p], gsem[p])
            pltpu.async_copy(
                bases[p].at[pl.ds(i_nxt * N + n0 + row0, GR)], bb[p], bsem[p])

        def wait(p):
            pltpu.make_async_copy(tabs[p].at[idx2[0]], gb[p], gsem[p]).wait()
            pltpu.make_async_copy(
                bases[p].at[pl.ds(0, GR)], bb[p], bsem[p]).wait()

        def group_code(t, i, g, buf, res_carry):
            row0 = g * GR
            wrap = g == ngrp - 1
            gn = (g + 1) & (ngrp - 1)
            i_n = jnp.where(wrap, (t + 1) & (M - 1), i)
            not_last = jnp.logical_not(
                jnp.logical_and(t == nsweep - 1, wrap))
            build_idx(buf, gn * GR, i_n * (M * K))

            res = res_carry
            for p in range(_NPH):
                wait(p)

                def n_body(nn, r, p=p):
                    if p == 0:
                        mv0, mi0 = inf16, zero16
                    else:
                        mv0 = mv_s[nn, :]
                        mi0 = mi_s[nn, :]

                    def s_body(sg, mc, p=p, nn=nn):
                        mv, mi = mc
                        a = bb[p][nn, pl.ds(sg * _L, _L)]
                        for j in range(M):
                            a = a + gb[p][j * GR + nn, pl.ds(sg * _L, _L)]
                        kv = (p * KP) + sg * _L + iota
                        pr = a < mv
                        return (jnp.where(pr, a, mv),
                                jnp.where(pr, kv, mi))

                    mv, mi = lax.fori_loop(0, KP // _L, s_body, (mv0, mi0))
                    if p < _NPH - 1:
                        mv_s[nn, :] = mv
                        mi_s[nn, :] = mi
                        return r
                    for sh in (8, 4, 2, 1):
                        perm = iota ^ sh
                        mv2 = mv.at[perm].get(mode="promise_in_bounds")
                        mi2 = mi.at[perm].get(mode="promise_in_bounds")
                        tk = (mv2 < mv) | ((mv2 == mv) & (mi2 < mi))
                        mv = jnp.where(tk, mv2, mv)
                        mi = jnp.where(tk, mi2, mi)
                    return jnp.where(iota == nn, mi, r)

                res = lax.fori_loop(0, GR, n_body, res)

                @pl.when(not_last)
                def _(p=p):
                    issue(p, buf, gn * GR, i_n)

            b_loc[pl.ds(i * rows_per_w + row0, _L)] = res

        build_idx(idx2[1], 0, 0)
        for p in range(_NPH):
            issue(p, idx2[1], 0, 0)

        def sweep_body(t, carry):
            i = t & (M - 1)

            def pair_body(q, carry2):
                group_code(t, i, 2 * q, idx2[0], zero16)
                group_code(t, i, 2 * q + 1, idx2[1], zero16)
                return carry2

            return lax.fori_loop(0, ngrp // 2, pair_body, carry)

        lax.fori_loop(0, nsweep, sweep_body, 0)
        for j in range(M):
            pltpu.sync_copy(b_loc.at[pl.ds(j * rows_per_w, rows_per_w)],
                            out_hbm.at[pl.ds(j * N + n0, rows_per_w)])

    return icm


def kernel(X, B, C_unary, C_pair, C):
    N, D = X.shape
    M = B.shape[1]
    K = C_pair.shape[2]

    bases = _make_base(N, M, K, D, BN=512)(X, C, C_unary)
    tabs = _make_transpose(M * M, K)(C_pair.reshape(M * M, K, K))
    rows_per_w = N // _NW
    out = _make_icm(N, M, K, 2, rows_per_w)(
        *bases, *tabs, B.T.reshape(-1))
    return out.reshape(M, N).T
